# Initial kernel scaffold; baseline (speedup 1.0000x reference)
import jax
import jax.numpy as jnp
from jax.experimental import pallas as pl


def kernel(x, Wc, bc, Wp, bp, Wt, bt, cb_c, cb_p, cb_t, Wr, br):
    raise NotImplementedError("write your pallas kernel here")


# speedup vs baseline: 2.2360x; 2.2360x over previous
import functools

import jax
import jax.numpy as jnp
from jax import lax
from jax.experimental import pallas as pl
from jax.experimental.pallas import tpu as pltpu

_B = 16
_T = 1024
_N = _B * _T
_IN = 512
_LAT = 256
_TN = 256


def _factor(x, W, b, cb):
    z = jnp.dot(x, W, preferred_element_type=jnp.float32) + b
    zn = jnp.sum(z * z, axis=1, keepdims=True)
    cbn = jnp.sum(cb * cb, axis=1)
    mm = lax.dot_general(z, cb, (((1,), (1,)), ((), ())),
                         preferred_element_type=jnp.float32)
    d = zn + cbn - 2.0 * mm
    dmin = jnp.min(d, axis=1, keepdims=True)
    jj = lax.broadcasted_iota(jnp.int32, d.shape, 1)
    idx = jnp.min(jnp.where(d == dmin, jj, d.shape[1]), axis=1)
    oh = (jj == idx[:, None]).astype(jnp.float32)
    zq = jnp.dot(oh, cb, preferred_element_type=jnp.float32)
    return idx, zq, jnp.sum(dmin)


def _body(x_ref, Wc_ref, bc_ref, Wp_ref, bp_ref, Wt_ref, bt_ref,
          cbc_ref, cbp_ref, cbt_ref, Wr_ref, br_ref,
          xr_ref, ci_ref, pi_ref, ti_ref, zcq_ref, zpq_ref, ztq_ref,
          parts_ref):
    x = x_ref[...]
    ic, zcq, s_c = _factor(x, Wc_ref[...], bc_ref[...], cbc_ref[...])
    ip, zpq, s_p = _factor(x, Wp_ref[...], bp_ref[...], cbp_ref[...])
    it, ztq, s_t = _factor(x, Wt_ref[...], bt_ref[...], cbt_ref[...])
    zq = jnp.concatenate([zcq, zpq, ztq], axis=1)
    xr_ref[...] = (jnp.dot(zq, Wr_ref[...], preferred_element_type=jnp.float32)
                   + br_ref[...])
    ci_ref[...] = ic.reshape(1, 1, _TN)
    pi_ref[...] = ip.reshape(1, 1, _TN)
    ti_ref[...] = it.reshape(1, 1, _TN)
    zcq_ref[...] = zcq
    zpq_ref[...] = zpq
    ztq_ref[...] = ztq
    rr = lax.broadcasted_iota(jnp.int32, (3, 128), 0)
    parts = jnp.where(rr == 0, s_c, jnp.where(rr == 1, s_p, s_t))
    parts_ref[...] = parts.reshape(1, 3, 128)


@jax.jit
def kernel(x, Wc, bc, Wp, bp, Wt, bt, cb_c, cb_p, cb_t, Wr, br):
    g = _N // _TN
    xf = x.reshape(_N, _IN)
    full = lambda shape: pl.BlockSpec(shape, lambda i: (0,) * len(shape))
    out_shapes = (
        jax.ShapeDtypeStruct((_N, _IN), jnp.float32),
        jax.ShapeDtypeStruct((g, 1, _TN), jnp.int32),
        jax.ShapeDtypeStruct((g, 1, _TN), jnp.int32),
        jax.ShapeDtypeStruct((g, 1, _TN), jnp.int32),
        jax.ShapeDtypeStruct((_N, _LAT), jnp.float32),
        jax.ShapeDtypeStruct((_N, _LAT), jnp.float32),
        jax.ShapeDtypeStruct((_N, _LAT), jnp.float32),
        jax.ShapeDtypeStruct((g, 3, 128), jnp.float32),
    )
    in_specs = [
        pl.BlockSpec((_TN, _IN), lambda i: (i, 0)),
        full((_IN, _LAT)), full((1, _LAT)),
        full((_IN, _LAT)), full((1, _LAT)),
        full((_IN, _LAT)), full((1, _LAT)),
        full((1024, _LAT)), full((1024, _LAT)), full((512, _LAT)),
        full((3 * _LAT, _IN)), full((1, _IN)),
    ]
    out_specs = (
        pl.BlockSpec((_TN, _IN), lambda i: (i, 0)),
        pl.BlockSpec((1, 1, _TN), lambda i: (i, 0, 0)),
        pl.BlockSpec((1, 1, _TN), lambda i: (i, 0, 0)),
        pl.BlockSpec((1, 1, _TN), lambda i: (i, 0, 0)),
        pl.BlockSpec((_TN, _LAT), lambda i: (i, 0)),
        pl.BlockSpec((_TN, _LAT), lambda i: (i, 0)),
        pl.BlockSpec((_TN, _LAT), lambda i: (i, 0)),
        pl.BlockSpec((1, 3, 128), lambda i: (i, 0, 0)),
    )
    outs = pl.pallas_call(
        _body,
        grid=(g,),
        in_specs=in_specs,
        out_specs=out_specs,
        out_shape=out_shapes,
        compiler_params=pltpu.CompilerParams(
            dimension_semantics=("arbitrary",)),
    )(xf, Wc, bc.reshape(1, _LAT), Wp, bp.reshape(1, _LAT),
      Wt, bt.reshape(1, _LAT), cb_c, cb_p, cb_t, Wr, br.reshape(1, _IN))
    xr, ci, pi, ti, zcq, zpq, ztq, parts = outs
    sums = parts[:, :, 0].sum(axis=0)
    mse_mean = (sums[0] + sums[1] + sums[2]) / (3.0 * _N * _LAT)
    vq_loss = mse_mean + 0.25 * mse_mean
    return (xr.reshape(_B, _T, _IN), vq_loss,
            ci.reshape(_B, _T), pi.reshape(_B, _T), ti.reshape(_B, _T),
            zcq.reshape(_B, _T, _LAT), zpq.reshape(_B, _T, _LAT),
            ztq.reshape(_B, _T, _LAT))


# speedup vs baseline: 2.3092x; 1.0327x over previous
import functools

import jax
import jax.numpy as jnp
from jax import lax
from jax.experimental import pallas as pl
from jax.experimental.pallas import tpu as pltpu

_B = 16
_T = 1024
_N = _B * _T
_IN = 512
_LAT = 256
_TN = 256


def _factor(z, cb, cb16):
    zn = jnp.sum(z * z, axis=1, keepdims=True)
    cbn = jnp.sum(cb * cb, axis=1)
    mm = lax.dot_general(z, cb, (((1,), (1,)), ((), ())),
                         preferred_element_type=jnp.float32)
    d = zn + cbn - 2.0 * mm
    dmin = jnp.min(d, axis=1, keepdims=True)
    jj = lax.broadcasted_iota(jnp.int32, d.shape, 1)
    idx = jnp.min(jnp.where(d == dmin, jj, d.shape[1]), axis=1)
    oh = (jj == idx[:, None]).astype(jnp.bfloat16)
    zq = jnp.dot(oh, cb16, preferred_element_type=jnp.float32)
    return idx, zq, jnp.sum(dmin)


def _body(x_ref, Wall_ref, ball_ref,
          cbc_ref, cbp_ref, cbt_ref, cbc16_ref, cbp16_ref, cbt16_ref,
          Wr16_ref, br_ref,
          xr_ref, ci_ref, pi_ref, ti_ref, zcq_ref, zpq_ref, ztq_ref,
          parts_ref):
    x = x_ref[...]
    z_all = jnp.dot(x, Wall_ref[...],
                    preferred_element_type=jnp.float32) + ball_ref[...]
    ic, zcq, s_c = _factor(z_all[:, :_LAT], cbc_ref[...], cbc16_ref[...])
    ip, zpq, s_p = _factor(z_all[:, _LAT:2 * _LAT], cbp_ref[...], cbp16_ref[...])
    it, ztq, s_t = _factor(z_all[:, 2 * _LAT:], cbt_ref[...], cbt16_ref[...])
    zq = jnp.concatenate([zcq, zpq, ztq], axis=1).astype(jnp.bfloat16)
    xr_ref[...] = (jnp.dot(zq, Wr16_ref[...], preferred_element_type=jnp.float32)
                   + br_ref[...])
    ci_ref[...] = ic.reshape(1, 1, _TN)
    pi_ref[...] = ip.reshape(1, 1, _TN)
    ti_ref[...] = it.reshape(1, 1, _TN)
    zcq_ref[...] = zcq
    zpq_ref[...] = zpq
    ztq_ref[...] = ztq
    rr = lax.broadcasted_iota(jnp.int32, (3, 128), 0)
    parts = jnp.where(rr == 0, s_c, jnp.where(rr == 1, s_p, s_t))
    parts_ref[...] = parts.reshape(1, 3, 128)


@jax.jit
def kernel(x, Wc, bc, Wp, bp, Wt, bt, cb_c, cb_p, cb_t, Wr, br):
    g = _N // _TN
    xf = x.reshape(_N, _IN)
    full = lambda shape: pl.BlockSpec(shape, lambda i: (0,) * len(shape))
    out_shapes = (
        jax.ShapeDtypeStruct((_N, _IN), jnp.float32),
        jax.ShapeDtypeStruct((g, 1, _TN), jnp.int32),
        jax.ShapeDtypeStruct((g, 1, _TN), jnp.int32),
        jax.ShapeDtypeStruct((g, 1, _TN), jnp.int32),
        jax.ShapeDtypeStruct((_N, _LAT), jnp.float32),
        jax.ShapeDtypeStruct((_N, _LAT), jnp.float32),
        jax.ShapeDtypeStruct((_N, _LAT), jnp.float32),
        jax.ShapeDtypeStruct((g, 3, 128), jnp.float32),
    )
    in_specs = [
        pl.BlockSpec((_TN, _IN), lambda i: (i, 0)),
        full((_IN, 3 * _LAT)), full((1, 3 * _LAT)),
        full((1024, _LAT)), full((1024, _LAT)), full((512, _LAT)),
        full((1024, _LAT)), full((1024, _LAT)), full((512, _LAT)),
        full((3 * _LAT, _IN)), full((1, _IN)),
    ]
    out_specs = (
        pl.BlockSpec((_TN, _IN), lambda i: (i, 0)),
        pl.BlockSpec((1, 1, _TN), lambda i: (i, 0, 0)),
        pl.BlockSpec((1, 1, _TN), lambda i: (i, 0, 0)),
        pl.BlockSpec((1, 1, _TN), lambda i: (i, 0, 0)),
        pl.BlockSpec((_TN, _LAT), lambda i: (i, 0)),
        pl.BlockSpec((_TN, _LAT), lambda i: (i, 0)),
        pl.BlockSpec((_TN, _LAT), lambda i: (i, 0)),
        pl.BlockSpec((1, 3, 128), lambda i: (i, 0, 0)),
    )
    outs = pl.pallas_call(
        _body,
        grid=(g,),
        in_specs=in_specs,
        out_specs=out_specs,
        out_shape=out_shapes,
        compiler_params=pltpu.CompilerParams(
            dimension_semantics=("arbitrary",)),
    )(xf,
      jnp.concatenate([Wc, Wp, Wt], axis=1),
      jnp.concatenate([bc, bp, bt]).reshape(1, 3 * _LAT),
      cb_c, cb_p, cb_t,
      cb_c.astype(jnp.bfloat16), cb_p.astype(jnp.bfloat16),
      cb_t.astype(jnp.bfloat16),
      Wr.astype(jnp.bfloat16), br.reshape(1, _IN))
    xr, ci, pi, ti, zcq, zpq, ztq, parts = outs
    sums = parts[:, :, 0].sum(axis=0)
    mse_mean = (sums[0] + sums[1] + sums[2]) / (3.0 * _N * _LAT)
    vq_loss = mse_mean + 0.25 * mse_mean
    return (xr.reshape(_B, _T, _IN), vq_loss,
            ci.reshape(_B, _T), pi.reshape(_B, _T), ti.reshape(_B, _T),
            zcq.reshape(_B, _T, _LAT), zpq.reshape(_B, _T, _LAT),
            ztq.reshape(_B, _T, _LAT))


# speedup vs baseline: 3.0364x; 1.3149x over previous
import functools

import jax
import jax.numpy as jnp
from jax import lax
from jax.experimental import pallas as pl
from jax.experimental.pallas import tpu as pltpu

_B = 16
_T = 1024
_N = _B * _T
_IN = 512
_LAT = 256
_TN = 512


def _factor(z, z2, cbn_row, jjf, cb, cb16):
    zn = jnp.sum(z * z, axis=1, keepdims=True)
    mm2 = lax.dot_general(z2, cb, (((1,), (1,)), ((), ())),
                          preferred_element_type=jnp.float32)
    d = (zn + cbn_row) - mm2
    dmin = jnp.min(d, axis=1, keepdims=True)
    jv = jjf[:, :d.shape[1]]
    idxf = jnp.min(jnp.where(d == dmin, jv, float(d.shape[1])), axis=1)
    idx = idxf.astype(jnp.int32)
    oh = (jv == idxf[:, None]).astype(jnp.bfloat16)
    zq = jnp.dot(oh, cb16, preferred_element_type=jnp.float32)
    return idx, zq, jnp.sum(dmin)


def _body(x_ref, Wall_ref, ball_ref,
          cbc_ref, cbp_ref, cbt_ref, cbc16_ref, cbp16_ref, cbt16_ref,
          Wr16_ref, br_ref,
          xr_ref, ci_ref, pi_ref, ti_ref, zcq_ref, zpq_ref, ztq_ref,
          parts_ref, cbn_ref):
    @pl.when(pl.program_id(0) == 0)
    def _init():
        cbn_ref[0:1, :1024] = jnp.sum(cbc_ref[...] * cbc_ref[...],
                                    axis=1).reshape(1, 1024)
        cbn_ref[1:2, :1024] = jnp.sum(cbp_ref[...] * cbp_ref[...],
                                    axis=1).reshape(1, 1024)
        cbn_ref[2:3, :512] = jnp.sum(cbt_ref[...] * cbt_ref[...],
                                   axis=1).reshape(1, 512)

    x = x_ref[...]
    z_all = jnp.dot(x, Wall_ref[...],
                    preferred_element_type=jnp.float32) + ball_ref[...]
    z2_all = z_all + z_all
    jjf = lax.broadcasted_iota(jnp.int32, (_TN, 1024), 1).astype(jnp.float32)
    ic, zcq, s_c = _factor(z_all[:, :_LAT], z2_all[:, :_LAT],
                           cbn_ref[0:1, :1024], jjf,
                           cbc_ref[...], cbc16_ref[...])
    ip, zpq, s_p = _factor(z_all[:, _LAT:2 * _LAT], z2_all[:, _LAT:2 * _LAT],
                           cbn_ref[1:2, :1024], jjf,
                           cbp_ref[...], cbp16_ref[...])
    it, ztq, s_t = _factor(z_all[:, 2 * _LAT:], z2_all[:, 2 * _LAT:],
                           cbn_ref[2:3, :512], jjf,
                           cbt_ref[...], cbt16_ref[...])
    zq = jnp.concatenate([zcq, zpq, ztq], axis=1).astype(jnp.bfloat16)
    xr_ref[...] = (jnp.dot(zq, Wr16_ref[...], preferred_element_type=jnp.float32)
                   + br_ref[...])
    ci_ref[...] = ic.reshape(1, 1, _TN)
    pi_ref[...] = ip.reshape(1, 1, _TN)
    ti_ref[...] = it.reshape(1, 1, _TN)
    zcq_ref[...] = zcq
    zpq_ref[...] = zpq
    ztq_ref[...] = ztq
    rr = lax.broadcasted_iota(jnp.int32, (3, 128), 0)
    parts = jnp.where(rr == 0, s_c, jnp.where(rr == 1, s_p, s_t))
    parts_ref[...] = parts.reshape(1, 3, 128)


@jax.jit
def kernel(x, Wc, bc, Wp, bp, Wt, bt, cb_c, cb_p, cb_t, Wr, br):
    g = _N // _TN
    xf = x.reshape(_N, _IN)
    full = lambda shape: pl.BlockSpec(shape, lambda i: (0,) * len(shape))
    out_shapes = (
        jax.ShapeDtypeStruct((_N, _IN), jnp.float32),
        jax.ShapeDtypeStruct((g, 1, _TN), jnp.int32),
        jax.ShapeDtypeStruct((g, 1, _TN), jnp.int32),
        jax.ShapeDtypeStruct((g, 1, _TN), jnp.int32),
        jax.ShapeDtypeStruct((_N, _LAT), jnp.float32),
        jax.ShapeDtypeStruct((_N, _LAT), jnp.float32),
        jax.ShapeDtypeStruct((_N, _LAT), jnp.float32),
        jax.ShapeDtypeStruct((g, 3, 128), jnp.float32),
    )
    in_specs = [
        pl.BlockSpec((_TN, _IN), lambda i: (i, 0)),
        full((_IN, 3 * _LAT)), full((1, 3 * _LAT)),
        full((1024, _LAT)), full((1024, _LAT)), full((512, _LAT)),
        full((1024, _LAT)), full((1024, _LAT)), full((512, _LAT)),
        full((3 * _LAT, _IN)), full((1, _IN)),
    ]
    out_specs = (
        pl.BlockSpec((_TN, _IN), lambda i: (i, 0)),
        pl.BlockSpec((1, 1, _TN), lambda i: (i, 0, 0)),
        pl.BlockSpec((1, 1, _TN), lambda i: (i, 0, 0)),
        pl.BlockSpec((1, 1, _TN), lambda i: (i, 0, 0)),
        pl.BlockSpec((_TN, _LAT), lambda i: (i, 0)),
        pl.BlockSpec((_TN, _LAT), lambda i: (i, 0)),
        pl.BlockSpec((_TN, _LAT), lambda i: (i, 0)),
        pl.BlockSpec((1, 3, 128), lambda i: (i, 0, 0)),
    )
    outs = pl.pallas_call(
        _body,
        grid=(g,),
        in_specs=in_specs,
        out_specs=out_specs,
        out_shape=out_shapes,
        scratch_shapes=[pltpu.VMEM((3, 1024), jnp.float32)],
        compiler_params=pltpu.CompilerParams(
            dimension_semantics=("arbitrary",)),
    )(xf,
      jnp.concatenate([Wc, Wp, Wt], axis=1),
      jnp.concatenate([bc, bp, bt]).reshape(1, 3 * _LAT),
      cb_c, cb_p, cb_t,
      cb_c.astype(jnp.bfloat16), cb_p.astype(jnp.bfloat16),
      cb_t.astype(jnp.bfloat16),
      Wr.astype(jnp.bfloat16), br.reshape(1, _IN))
    xr, ci, pi, ti, zcq, zpq, ztq, parts = outs
    sums = parts[:, :, 0].sum(axis=0)
    mse_mean = (sums[0] + sums[1] + sums[2]) / (3.0 * _N * _LAT)
    vq_loss = mse_mean + 0.25 * mse_mean
    return (xr.reshape(_B, _T, _IN), vq_loss,
            ci.reshape(_B, _T), pi.reshape(_B, _T), ti.reshape(_B, _T),
            zcq.reshape(_B, _T, _LAT), zpq.reshape(_B, _T, _LAT),
            ztq.reshape(_B, _T, _LAT))


# speedup vs baseline: 3.1907x; 1.0508x over previous
import functools

import jax
import jax.numpy as jnp
from jax import lax
from jax.experimental import pallas as pl
from jax.experimental.pallas import tpu as pltpu

_B = 16
_T = 1024
_N = _B * _T
_IN = 512
_LAT = 256
_TN = 1024


def _factor(z, z2, cbn_row, jjf, cb, cb16):
    zn = jnp.sum(z * z, axis=1, keepdims=True)
    mm2 = lax.dot_general(z2, cb, (((1,), (1,)), ((), ())),
                          preferred_element_type=jnp.float32)
    d = (zn + cbn_row) - mm2
    dmin = jnp.min(d, axis=1, keepdims=True)
    jv = jjf[:, :d.shape[1]]
    idxf = jnp.min(jnp.where(d == dmin, jv, float(d.shape[1])), axis=1)
    idx = idxf.astype(jnp.int32)
    oh = (jv == idxf[:, None]).astype(jnp.bfloat16)
    zq = jnp.dot(oh, cb16, preferred_element_type=jnp.float32)
    return idx, zq, jnp.sum(dmin)


def _body(x_ref, Wall_ref, ball_ref,
          cbc_ref, cbp_ref, cbt_ref, cbc16_ref, cbp16_ref, cbt16_ref,
          Wr16_ref, br_ref,
          xr_ref, ci_ref, pi_ref, ti_ref, zcq_ref, zpq_ref, ztq_ref,
          parts_ref, cbn_ref):
    @pl.when(pl.program_id(0) == 0)
    def _init():
        cbn_ref[0:1, :1024] = jnp.sum(cbc_ref[...] * cbc_ref[...],
                                    axis=1).reshape(1, 1024)
        cbn_ref[1:2, :1024] = jnp.sum(cbp_ref[...] * cbp_ref[...],
                                    axis=1).reshape(1, 1024)
        cbn_ref[2:3, :512] = jnp.sum(cbt_ref[...] * cbt_ref[...],
                                   axis=1).reshape(1, 512)

    x = x_ref[...]
    z_all = jnp.dot(x, Wall_ref[...],
                    preferred_element_type=jnp.float32) + ball_ref[...]
    z2_all = z_all + z_all
    jjf = lax.broadcasted_iota(jnp.int32, (_TN, 1024), 1).astype(jnp.float32)
    ic, zcq, s_c = _factor(z_all[:, :_LAT], z2_all[:, :_LAT],
                           cbn_ref[0:1, :1024], jjf,
                           cbc_ref[...], cbc16_ref[...])
    ip, zpq, s_p = _factor(z_all[:, _LAT:2 * _LAT], z2_all[:, _LAT:2 * _LAT],
                           cbn_ref[1:2, :1024], jjf,
                           cbp_ref[...], cbp16_ref[...])
    it, ztq, s_t = _factor(z_all[:, 2 * _LAT:], z2_all[:, 2 * _LAT:],
                           cbn_ref[2:3, :512], jjf,
                           cbt_ref[...], cbt16_ref[...])
    zq = jnp.concatenate([zcq, zpq, ztq], axis=1).astype(jnp.bfloat16)
    xr_ref[...] = (jnp.dot(zq, Wr16_ref[...], preferred_element_type=jnp.float32)
                   + br_ref[...])
    ci_ref[...] = ic.reshape(1, 1, _TN)
    pi_ref[...] = ip.reshape(1, 1, _TN)
    ti_ref[...] = it.reshape(1, 1, _TN)
    zcq_ref[...] = zcq
    zpq_ref[...] = zpq
    ztq_ref[...] = ztq
    rr = lax.broadcasted_iota(jnp.int32, (3, 128), 0)
    parts = jnp.where(rr == 0, s_c, jnp.where(rr == 1, s_p, s_t))
    parts_ref[...] = parts.reshape(1, 3, 128)


@jax.jit
def kernel(x, Wc, bc, Wp, bp, Wt, bt, cb_c, cb_p, cb_t, Wr, br):
    g = _N // _TN
    xf = x.reshape(_N, _IN)
    full = lambda shape: pl.BlockSpec(shape, lambda i: (0,) * len(shape))
    out_shapes = (
        jax.ShapeDtypeStruct((_N, _IN), jnp.float32),
        jax.ShapeDtypeStruct((g, 1, _TN), jnp.int32),
        jax.ShapeDtypeStruct((g, 1, _TN), jnp.int32),
        jax.ShapeDtypeStruct((g, 1, _TN), jnp.int32),
        jax.ShapeDtypeStruct((_N, _LAT), jnp.float32),
        jax.ShapeDtypeStruct((_N, _LAT), jnp.float32),
        jax.ShapeDtypeStruct((_N, _LAT), jnp.float32),
        jax.ShapeDtypeStruct((g, 3, 128), jnp.float32),
    )
    in_specs = [
        pl.BlockSpec((_TN, _IN), lambda i: (i, 0)),
        full((_IN, 3 * _LAT)), full((1, 3 * _LAT)),
        full((1024, _LAT)), full((1024, _LAT)), full((512, _LAT)),
        full((1024, _LAT)), full((1024, _LAT)), full((512, _LAT)),
        full((3 * _LAT, _IN)), full((1, _IN)),
    ]
    out_specs = (
        pl.BlockSpec((_TN, _IN), lambda i: (i, 0)),
        pl.BlockSpec((1, 1, _TN), lambda i: (i, 0, 0)),
        pl.BlockSpec((1, 1, _TN), lambda i: (i, 0, 0)),
        pl.BlockSpec((1, 1, _TN), lambda i: (i, 0, 0)),
        pl.BlockSpec((_TN, _LAT), lambda i: (i, 0)),
        pl.BlockSpec((_TN, _LAT), lambda i: (i, 0)),
        pl.BlockSpec((_TN, _LAT), lambda i: (i, 0)),
        pl.BlockSpec((1, 3, 128), lambda i: (i, 0, 0)),
    )
    outs = pl.pallas_call(
        _body,
        grid=(g,),
        in_specs=in_specs,
        out_specs=out_specs,
        out_shape=out_shapes,
        scratch_shapes=[pltpu.VMEM((3, 1024), jnp.float32)],
        compiler_params=pltpu.CompilerParams(
            dimension_semantics=("arbitrary",)),
    )(xf,
      jnp.concatenate([Wc, Wp, Wt], axis=1),
      jnp.concatenate([bc, bp, bt]).reshape(1, 3 * _LAT),
      cb_c, cb_p, cb_t,
      cb_c.astype(jnp.bfloat16), cb_p.astype(jnp.bfloat16),
      cb_t.astype(jnp.bfloat16),
      Wr.astype(jnp.bfloat16), br.reshape(1, _IN))
    xr, ci, pi, ti, zcq, zpq, ztq, parts = outs
    sums = parts[:, :, 0].sum(axis=0)
    mse_mean = (sums[0] + sums[1] + sums[2]) / (3.0 * _N * _LAT)
    vq_loss = mse_mean + 0.25 * mse_mean
    return (xr.reshape(_B, _T, _IN), vq_loss,
            ci.reshape(_B, _T), pi.reshape(_B, _T), ti.reshape(_B, _T),
            zcq.reshape(_B, _T, _LAT), zpq.reshape(_B, _T, _LAT),
            ztq.reshape(_B, _T, _LAT))
